# baseline (speedup 1.0000x reference)
import jax
import jax.numpy as jnp
from jax.experimental import pallas as pl
from jax.experimental.pallas import tpu as pltpu

_BLK = 256


def _pool_kernel(adj_ref, x_ref, adjo_ref, xo_ref, sem):
    @pl.when(pl.program_id(0) == 0)
    def _start():
        pltpu.make_async_copy(adj_ref, adjo_ref, sem).start()

    xo_ref[...] = x_ref[...] + x_ref[...]

    @pl.when(pl.program_id(0) == pl.num_programs(0) - 1)
    def _wait():
        pltpu.make_async_copy(adj_ref, adjo_ref, sem).wait()


def kernel(x, adj, W, b):
    n, d = x.shape
    xspec = pl.BlockSpec((_BLK, d), lambda i: (i, 0))
    anyspec = pl.BlockSpec(memory_space=pl.ANY)
    adj_out, x_new = pl.pallas_call(
        _pool_kernel,
        grid=(n // _BLK,),
        in_specs=[anyspec, xspec],
        out_specs=[anyspec, xspec],
        out_shape=(
            jax.ShapeDtypeStruct(adj.shape, adj.dtype),
            jax.ShapeDtypeStruct((n, d), x.dtype),
        ),
        scratch_shapes=[pltpu.SemaphoreType.DMA],
    )(adj, x)
    return (x_new, adj_out)


# speedup vs baseline: 17.3422x; 17.3422x over previous
import jax
import jax.numpy as jnp
from jax.experimental import pallas as pl

_BLK = 1024


def _pool_kernel(x_ref, adj_ref, xo_ref, adjo_ref):
    xo_ref[...] = x_ref[...] + x_ref[...]
    adjo_ref[...] = adj_ref[...]


def kernel(x, adj, W, b):
    n, d = x.shape
    grid = (n // _BLK,)
    spec = pl.BlockSpec((_BLK, d), lambda i: (i, 0))
    x_new, adj_out = pl.pallas_call(
        _pool_kernel,
        grid=grid,
        in_specs=[spec, spec],
        out_specs=[spec, spec],
        out_shape=(
            jax.ShapeDtypeStruct((n, d), x.dtype),
            jax.ShapeDtypeStruct(adj.shape, adj.dtype),
        ),
    )(x, adj)
    return (x_new, adj_out)


# speedup vs baseline: 20.0902x; 1.1585x over previous
import jax
import jax.numpy as jnp
from jax.experimental import pallas as pl

_BLK = 512


def _pool_kernel(x_ref, adj_ref, xo_ref, adjo_ref):
    xo_ref[...] = x_ref[...] + x_ref[...]
    adjo_ref[...] = adj_ref[...]


def kernel(x, adj, W, b):
    n, d = x.shape
    grid = (n // _BLK,)
    spec = pl.BlockSpec((_BLK, d), lambda i: (i, 0))
    x_new, adj_out = pl.pallas_call(
        _pool_kernel,
        grid=grid,
        in_specs=[spec, spec],
        out_specs=[spec, spec],
        out_shape=(
            jax.ShapeDtypeStruct((n, d), x.dtype),
            jax.ShapeDtypeStruct(adj.shape, adj.dtype),
        ),
    )(x, adj)
    return (x_new, adj_out)
